# baseline (speedup 1.0000x reference)
import functools

import jax
import jax.numpy as jnp
from jax.experimental import pallas as pl
from jax.experimental.pallas import tpu as pltpu

IOU_THRESHOLD = 0.6
NMS_FILTER = 2000
BLK = 128
CHUNK = 16
PAD_COORD = -1.0e6


def _nms_block_kernel(x1_ref, y1_ref, x2_ref, y2_ref,
                      keep_ref, removed_ref, m_ref, ks_ref, nblk: int):
    b = pl.program_id(0)

    @pl.when(b == 0)
    def _init():
        removed_ref[...] = jnp.zeros_like(removed_ref)

    cx1 = x1_ref[pl.ds(b, 1), :]
    cy1 = y1_ref[pl.ds(b, 1), :]
    cx2 = x2_ref[pl.ds(b, 1), :]
    cy2 = y2_ref[pl.ds(b, 1), :]
    rx1 = jnp.broadcast_to(cx1, (BLK, BLK)).T
    ry1 = jnp.broadcast_to(cy1, (BLK, BLK)).T
    rx2 = jnp.broadcast_to(cx2, (BLK, BLK)).T
    ry2 = jnp.broadcast_to(cy2, (BLK, BLK)).T
    rarea = (rx2 - rx1) * (ry2 - ry1)

    def iou_gt(ccx1, ccy1, ccx2, ccy2):
        ix1 = jnp.maximum(rx1, ccx1)
        iy1 = jnp.maximum(ry1, ccy1)
        ix2 = jnp.minimum(rx2, ccx2)
        iy2 = jnp.minimum(ry2, ccy2)
        inter = jnp.maximum(ix2 - ix1, 0.0) * jnp.maximum(iy2 - iy1, 0.0)
        carea = (ccx2 - ccx1) * (ccy2 - ccy1)
        iou = inter / (rarea + carea - inter + 1e-9)
        return (iou > IOU_THRESHOLD).astype(jnp.float32)

    m = iou_gt(cx1, cy1, cx2, cy2)
    col_ids = jax.lax.broadcasted_iota(jnp.int32, (BLK, BLK), 1)
    row_ids = jax.lax.broadcasted_iota(jnp.int32, (BLK, BLK), 0)
    m_ref[...] = m * (col_ids > row_ids).astype(jnp.float32)

    lane = jax.lax.broadcasted_iota(jnp.int32, (1, BLK), 1)
    ks_ref[...] = 1.0 - removed_ref[pl.ds(b, 1), :]

    def chunk_body(k, _):
        base = pl.multiple_of(k * CHUNK, CHUNK)
        flag = jnp.max(m_ref[pl.ds(base, CHUNK), :])

        @pl.when(flag > 0.0)
        def _active():
            def body(i, _):
                mrow = m_ref[pl.ds(i, 1), :]
                keep = ks_ref[...]
                keep_i = jnp.sum(jnp.where(lane == i, keep, 0.0))
                ks_ref[...] = keep * (1.0 - mrow * keep_i)
                return 0
            jax.lax.fori_loop(base, base + CHUNK, body, 0)

        return 0

    jax.lax.fori_loop(0, BLK // CHUNK, chunk_body, 0)
    keep = ks_ref[...]
    keep_ref[pl.ds(b, 1), :] = keep

    keep_rows = jnp.broadcast_to(keep, (BLK, BLK)).T
    min_y1_b = jnp.min(cy1)

    def cross(c, _):
        ccy2 = y2_ref[pl.ds(c, 1), :]

        @pl.when(jnp.max(ccy2) > min_y1_b)
        def _work():
            ccx1 = x1_ref[pl.ds(c, 1), :]
            ccy1 = y1_ref[pl.ds(c, 1), :]
            ccx2 = x2_ref[pl.ds(c, 1), :]
            mc = iou_gt(ccx1, ccy1, ccx2, ccy2) * keep_rows
            sup = jnp.max(mc, axis=0, keepdims=True)
            removed_ref[pl.ds(c, 1), :] = jnp.maximum(
                removed_ref[pl.ds(c, 1), :], sup)

        return 0

    jax.lax.fori_loop(b + 1, nblk, cross, 0)


def _run_nms(bs, n_pad, interpret=False):
    nblk = n_pad // BLK
    planes = [bs[:, i].reshape(nblk, BLK) for i in range(4)]
    keep = pl.pallas_call(
        functools.partial(_nms_block_kernel, nblk=nblk),
        grid=(nblk,),
        in_specs=[pl.BlockSpec((nblk, BLK), lambda b: (0, 0))] * 4,
        out_specs=pl.BlockSpec((nblk, BLK), lambda b: (0, 0)),
        out_shape=jax.ShapeDtypeStruct((nblk, BLK), jnp.float32),
        scratch_shapes=[pltpu.VMEM((nblk, BLK), jnp.float32),
                        pltpu.VMEM((BLK, BLK), jnp.float32),
                        pltpu.VMEM((1, BLK), jnp.float32)],
        interpret=interpret,
    )(*planes)
    return keep.reshape(-1)


def kernel(anchors, scores):
    n = anchors.shape[0]
    n_pad = ((n + BLK - 1) // BLK) * BLK

    scores_fg = scores.reshape(-1, 2)[:, 1]
    top_scores_idx = jnp.argsort(scores_fg)
    top_anchors = anchors[top_scores_idx]
    top_scores = top_anchors[:, 1]
    order = jnp.argsort(-top_scores)
    bs = top_anchors[order]
    bs_pad = jnp.concatenate(
        [bs, jnp.full((n_pad - n, 4), PAD_COORD, jnp.float32)], axis=0)

    keep = _run_nms(bs_pad, n_pad)[:n] > 0.5

    sel = jnp.nonzero(keep, size=min(NMS_FILTER, n), fill_value=0)[0]
    nms_idx = order[sel]
    return anchors[nms_idx], nms_idx


# speedup vs baseline: 1.3769x; 1.3769x over previous
import functools

import jax
import jax.numpy as jnp
from jax import lax
from jax.experimental import pallas as pl
from jax.experimental.pallas import tpu as pltpu
from jax.experimental.pallas import tpu_sc as plsc

IOU_THRESHOLD = 0.6
NMS_FILTER = 2000
BLK = 128
PAD_COORD = -1.0e6

NWORK = 32
LANES = 16
NOUT = 2048


def _nms_block_kernel(x1_ref, y1_ref, x2_ref, y2_ref,
                      keep_ref, bsum_ref, removed_ref, m_ref, suff_ref,
                      nblk: int):
    b = pl.program_id(0)

    @pl.when(b == 0)
    def _init():
        removed_ref[...] = jnp.zeros_like(removed_ref)

        def sm_body(i, carry):
            c = nblk - 1 - i
            new = jnp.maximum(jnp.max(y2_ref[pl.ds(c, 1), :]), carry)
            suff_ref[c] = new
            return new

        lax.fori_loop(0, nblk, sm_body, jnp.float32(-3e38))

    cx1 = x1_ref[pl.ds(b, 1), :]
    cy1 = y1_ref[pl.ds(b, 1), :]
    cx2 = x2_ref[pl.ds(b, 1), :]
    cy2 = y2_ref[pl.ds(b, 1), :]
    rx1 = jnp.broadcast_to(cx1, (BLK, BLK)).T
    ry1 = jnp.broadcast_to(cy1, (BLK, BLK)).T
    rx2 = jnp.broadcast_to(cx2, (BLK, BLK)).T
    ry2 = jnp.broadcast_to(cy2, (BLK, BLK)).T
    rarea = (rx2 - rx1) * (ry2 - ry1)

    def iou_gt(ccx1, ccy1, ccx2, ccy2):
        ix1 = jnp.maximum(rx1, ccx1)
        iy1 = jnp.maximum(ry1, ccy1)
        ix2 = jnp.minimum(rx2, ccx2)
        iy2 = jnp.minimum(ry2, ccy2)
        inter = jnp.maximum(ix2 - ix1, 0.0) * jnp.maximum(iy2 - iy1, 0.0)
        carea = (ccx2 - ccx1) * (ccy2 - ccy1)
        iou = inter / (rarea + carea - inter + 1e-9)
        return (iou > IOU_THRESHOLD).astype(jnp.float32)

    m = iou_gt(cx1, cy1, cx2, cy2)
    col_ids = lax.broadcasted_iota(jnp.int32, (BLK, BLK), 1)
    row_ids = lax.broadcasted_iota(jnp.int32, (BLK, BLK), 0)
    m_ref[...] = m * (col_ids > row_ids).astype(jnp.float32)

    lane = lax.broadcasted_iota(jnp.int32, (1, BLK), 1)
    keep0 = 1.0 - removed_ref[pl.ds(b, 1), :]

    def body(i, keep):
        mrow = m_ref[pl.ds(i, 1), :]
        keep_i = jnp.sum(jnp.where(lane == i, keep, 0.0))
        return keep * (1.0 - mrow * keep_i)

    keep = lax.fori_loop(0, BLK, body, keep0)
    keep_real = keep * (cy1 > -1.0e5).astype(jnp.float32)
    keep_ref[pl.ds(b, 1), :] = keep_real
    bsum_ref[pl.ds(b, 1), :] = jnp.broadcast_to(jnp.sum(keep_real), (1, BLK))

    keep_rows = jnp.broadcast_to(keep, (BLK, BLK)).T
    min_y1_b = jnp.min(cy1)

    c_hi = lax.while_loop(
        lambda c: (c < nblk) & (suff_ref[c] > min_y1_b),
        lambda c: c + 1, b + 1)

    def cross(c, _):
        ccx1 = x1_ref[pl.ds(c, 1), :]
        ccy1 = y1_ref[pl.ds(c, 1), :]
        ccx2 = x2_ref[pl.ds(c, 1), :]
        ccy2 = y2_ref[pl.ds(c, 1), :]
        mc = iou_gt(ccx1, ccy1, ccx2, ccy2) * keep_rows
        sup = jnp.max(mc, axis=0, keepdims=True)
        removed_ref[pl.ds(c, 1), :] = jnp.maximum(
            removed_ref[pl.ds(c, 1), :], sup)
        return 0

    lax.fori_loop(b + 1, c_hi, cross, 0)


def _run_nms(bs, n_pad, interpret=False):
    nblk = n_pad // BLK
    planes = [bs[:, i].reshape(nblk, BLK) for i in range(4)]
    keep, bsum = pl.pallas_call(
        functools.partial(_nms_block_kernel, nblk=nblk),
        grid=(nblk,),
        in_specs=[pl.BlockSpec((nblk, BLK), lambda b: (0, 0))] * 4,
        out_specs=[pl.BlockSpec((nblk, BLK), lambda b: (0, 0))] * 2,
        out_shape=[jax.ShapeDtypeStruct((nblk, BLK), jnp.float32),
                   jax.ShapeDtypeStruct((nblk, BLK), jnp.float32)],
        scratch_shapes=[pltpu.VMEM((nblk, BLK), jnp.float32),
                        pltpu.VMEM((BLK, BLK), jnp.float32),
                        pltpu.SMEM((nblk,), jnp.float32)],
        interpret=interpret,
    )(*planes)
    return keep.reshape(-1), bsum[:, 0]


def _fin_kernel(keep_hbm, vals_hbm, anch_hbm, wpre_hbm, out_hbm,
                keep_v, vals_v, slots_v, rows_v, wpre_v, tmp_v, sem,
                n_pad: int):
    ch = n_pad // NWORK
    ng = ch // LANES
    nc = ch // 128
    c = lax.axis_index("c")
    s = lax.axis_index("s")
    wid = s * 2 + c
    base = wid * ch

    pltpu.sync_copy(keep_hbm.at[pl.ds(base, ch)], keep_v)
    pltpu.sync_copy(vals_hbm.at[pl.ds(base, ch)], vals_v)
    pltpu.sync_copy(wpre_hbm, wpre_v)

    lanes = lax.iota(jnp.int32, LANES)
    last = lanes * 0 + (LANES - 1)
    carry = plsc.load_gather(wpre_v, [lanes * 0 + wid])

    for g in range(ng):
        kv = keep_v[pl.ds(g * LANES, LANES)]
        cur = kv
        for sh in (1, 2, 4, 8):
            tmp_v[pl.ds(0, LANES)] = cur
            sft = plsc.load_gather(tmp_v, [jnp.maximum(lanes - sh, 0)])
            cur = cur + jnp.where(lanes >= sh, sft, 0.0)
        tmp_v[pl.ds(0, LANES)] = cur
        tot = plsc.load_gather(tmp_v, [last])
        slot = (carry + cur - 1.0).astype(jnp.int32)
        ok = (kv > 0.0) & (slot < NOUT)
        slots_v[g // 8, pl.ds((g % 8) * LANES, LANES)] = (
            jnp.where(ok, slot, NOUT + wid))
        carry = carry + tot

    for j in range(nc):
        pltpu.async_copy(anch_hbm.at[vals_v.at[pl.ds(j * 128, 128)]],
                         rows_v.at[pl.ds(j * 128, 128)], sem).wait()

    zeros = jnp.zeros((LANES,), jnp.int32)

    def wcol(g, _):
        rid = lanes + g * LANES
        v = vals_v[pl.ds(g * LANES, LANES)].astype(jnp.float32)
        plsc.store_scatter(rows_v, [rid, zeros], v)
        return 0

    lax.fori_loop(0, ng, wcol, 0)

    for j in range(nc):
        pltpu.async_copy(rows_v.at[pl.ds(j * 128, 128)],
                         out_hbm.at[slots_v.at[j]], sem).wait()


def _finalize(keep, vals, anch16, wprefix, n_pad):
    ch = n_pad // NWORK
    mesh = plsc.VectorSubcoreMesh(core_axis_name="c", subcore_axis_name="s")
    out = pl.kernel(
        functools.partial(_fin_kernel, n_pad=n_pad),
        out_type=jax.ShapeDtypeStruct((NOUT + NWORK, 16), jnp.float32),
        mesh=mesh,
        compiler_params=pltpu.CompilerParams(needs_layout_passes=False,
                                             use_tc_tiling_on_sc=False),
        scratch_types=[
            pltpu.VMEM((ch,), jnp.float32),
            pltpu.VMEM((ch,), jnp.int32),
            pltpu.VMEM((ch // 128, 128), jnp.int32),
            pltpu.VMEM((ch, 16), jnp.float32),
            pltpu.VMEM((NWORK,), jnp.float32),
            pltpu.VMEM((LANES,), jnp.float32),
            pltpu.SemaphoreType.DMA,
        ],
    )(keep, vals, anch16, wprefix)
    return out


def kernel(anchors, scores):
    n = anchors.shape[0]
    grain = BLK * NWORK
    n_pad = ((n + grain - 1) // grain) * grain

    scores_fg = scores.reshape(-1, 2)[:, 1]
    top_scores_idx = jnp.argsort(scores_fg)
    top_anchors = anchors[top_scores_idx]
    top_scores = top_anchors[:, 1]
    order = jnp.argsort(-top_scores)
    bs = top_anchors[order]
    bs_pad = jnp.concatenate(
        [bs, jnp.full((n_pad - n, 4), PAD_COORD, jnp.float32)], axis=0)

    keep, bsum = _run_nms(bs_pad, n_pad)

    nblk = n_pad // BLK
    csum = jnp.cumsum(bsum)
    count = csum[-1].astype(jnp.int32)
    blocks_per_w = nblk // NWORK
    wprefix = jnp.concatenate(
        [jnp.zeros((1,), jnp.float32),
         csum[blocks_per_w - 1:-1:blocks_per_w]])

    vals = jnp.concatenate(
        [order.astype(jnp.int32), jnp.zeros((n_pad - n,), jnp.int32)])
    anch16 = jnp.pad(anchors, ((0, 0), (1, 11)))

    out = _finalize(keep, vals, anch16, wprefix, n_pad)

    valid = jnp.arange(NMS_FILTER) < count
    col0 = out[:NMS_FILTER, 0].astype(jnp.int32)
    nms_idx = jnp.where(valid, col0, col0[0])
    boxes = jnp.where(valid[:, None], out[:NMS_FILTER, 1:5], out[0:1, 1:5])
    return boxes, nms_idx


# speedup vs baseline: 1.6099x; 1.1693x over previous
import functools

import jax
import jax.numpy as jnp
from jax import lax
from jax.experimental import pallas as pl
from jax.experimental.pallas import tpu as pltpu
from jax.experimental.pallas import tpu_sc as plsc

IOU_THRESHOLD = 0.6
NMS_FILTER = 2000
BLK = 128
PAD_COORD = -1.0e6

NWORK = 32
LANES = 16
NOUT = 2048


def _nms_block_kernel(x1_ref, y1_ref, x2_ref, y2_ref,
                      keep_ref, bsum_ref, removed_ref, m_ref, suff_ref,
                      nblk: int):
    b = pl.program_id(0)

    @pl.when(b == 0)
    def _init():
        removed_ref[...] = jnp.zeros_like(removed_ref)

        def sm_body(i, carry):
            c = nblk - 1 - i
            new = jnp.maximum(jnp.max(y2_ref[pl.ds(c, 1), :]), carry)
            suff_ref[c] = new
            return new

        lax.fori_loop(0, nblk, sm_body, jnp.float32(-3e38))

    cx1 = x1_ref[pl.ds(b, 1), :]
    cy1 = y1_ref[pl.ds(b, 1), :]
    cx2 = x2_ref[pl.ds(b, 1), :]
    cy2 = y2_ref[pl.ds(b, 1), :]
    rx1 = jnp.broadcast_to(cx1, (BLK, BLK)).T
    ry1 = jnp.broadcast_to(cy1, (BLK, BLK)).T
    rx2 = jnp.broadcast_to(cx2, (BLK, BLK)).T
    ry2 = jnp.broadcast_to(cy2, (BLK, BLK)).T
    rarea = (rx2 - rx1) * (ry2 - ry1)

    def iou_gt(ccx1, ccy1, ccx2, ccy2):
        ix1 = jnp.maximum(rx1, ccx1)
        iy1 = jnp.maximum(ry1, ccy1)
        ix2 = jnp.minimum(rx2, ccx2)
        iy2 = jnp.minimum(ry2, ccy2)
        inter = jnp.maximum(ix2 - ix1, 0.0) * jnp.maximum(iy2 - iy1, 0.0)
        carea = (ccx2 - ccx1) * (ccy2 - ccy1)
        iou = inter / (rarea + carea - inter + 1e-9)
        return (iou > IOU_THRESHOLD).astype(jnp.float32)

    m = iou_gt(cx1, cy1, cx2, cy2)
    col_ids = lax.broadcasted_iota(jnp.int32, (BLK, BLK), 1)
    row_ids = lax.broadcasted_iota(jnp.int32, (BLK, BLK), 0)
    m_ref[...] = m * (col_ids > row_ids).astype(jnp.float32)

    lane = lax.broadcasted_iota(jnp.int32, (1, BLK), 1)
    keep0 = 1.0 - removed_ref[pl.ds(b, 1), :]

    def body(i, keep):
        mrow = m_ref[pl.ds(i, 1), :]
        keep_i = jnp.sum(jnp.where(lane == i, keep, 0.0))
        return keep * (1.0 - mrow * keep_i)

    CHUNK = 16

    def chunk_body(k, keep):
        base = pl.multiple_of(k * CHUNK, CHUNK)
        flag = jnp.max(m_ref[pl.ds(base, CHUNK), :])
        return lax.cond(
            flag > 0.0,
            lambda kp: lax.fori_loop(base, base + CHUNK, body, kp),
            lambda kp: kp,
            keep)

    keep = lax.fori_loop(0, BLK // CHUNK, chunk_body, keep0)
    keep_real = keep * (cy1 > -1.0e5).astype(jnp.float32)
    keep_ref[pl.ds(b, 1), :] = keep_real
    bsum_ref[pl.ds(b, 1), :] = jnp.broadcast_to(jnp.sum(keep_real), (1, BLK))

    keep_rows = jnp.broadcast_to(keep, (BLK, BLK)).T
    min_y1_b = jnp.min(cy1)

    c_hi = lax.while_loop(
        lambda c: (c < nblk) & (suff_ref[c] > min_y1_b),
        lambda c: c + 1, b + 1)

    def cross(c, _):
        ccx1 = x1_ref[pl.ds(c, 1), :]
        ccy1 = y1_ref[pl.ds(c, 1), :]
        ccx2 = x2_ref[pl.ds(c, 1), :]
        ccy2 = y2_ref[pl.ds(c, 1), :]
        mc = iou_gt(ccx1, ccy1, ccx2, ccy2) * keep_rows
        sup = jnp.max(mc, axis=0, keepdims=True)
        removed_ref[pl.ds(c, 1), :] = jnp.maximum(
            removed_ref[pl.ds(c, 1), :], sup)
        return 0

    lax.fori_loop(b + 1, c_hi, cross, 0)


def _run_nms(bs, n_pad, interpret=False):
    nblk = n_pad // BLK
    planes = [bs[:, i].reshape(nblk, BLK) for i in range(4)]
    keep, bsum = pl.pallas_call(
        functools.partial(_nms_block_kernel, nblk=nblk),
        grid=(nblk,),
        in_specs=[pl.BlockSpec((nblk, BLK), lambda b: (0, 0))] * 4,
        out_specs=[pl.BlockSpec((nblk, BLK), lambda b: (0, 0))] * 2,
        out_shape=[jax.ShapeDtypeStruct((nblk, BLK), jnp.float32),
                   jax.ShapeDtypeStruct((nblk, BLK), jnp.float32)],
        scratch_shapes=[pltpu.VMEM((nblk, BLK), jnp.float32),
                        pltpu.VMEM((BLK, BLK), jnp.float32),
                        pltpu.SMEM((nblk,), jnp.float32)],
        interpret=interpret,
    )(*planes)
    return keep.reshape(-1), bsum[:, 0]


def _fin_kernel(keep_hbm, vals_hbm, anch_hbm, wpre_hbm, out_hbm,
                keep_v, vals_v, slots_v, rows_v, wpre_v, tmp_v, sem,
                n_pad: int):
    ch = n_pad // NWORK
    ng = ch // LANES
    nc = ch // 128
    c = lax.axis_index("c")
    s = lax.axis_index("s")
    wid = s * 2 + c
    base = wid * ch

    pltpu.sync_copy(keep_hbm.at[pl.ds(base, ch)], keep_v)
    pltpu.sync_copy(vals_hbm.at[pl.ds(base, ch)], vals_v)
    pltpu.sync_copy(wpre_hbm, wpre_v)

    lanes = lax.iota(jnp.int32, LANES)
    last = lanes * 0 + (LANES - 1)
    carry = plsc.load_gather(wpre_v, [lanes * 0 + wid])

    for g in range(ng):
        kv = keep_v[pl.ds(g * LANES, LANES)]
        cur = kv
        for sh in (1, 2, 4, 8):
            tmp_v[pl.ds(0, LANES)] = cur
            sft = plsc.load_gather(tmp_v, [jnp.maximum(lanes - sh, 0)])
            cur = cur + jnp.where(lanes >= sh, sft, 0.0)
        tmp_v[pl.ds(0, LANES)] = cur
        tot = plsc.load_gather(tmp_v, [last])
        slot = (carry + cur - 1.0).astype(jnp.int32)
        ok = (kv > 0.0) & (slot < NOUT)
        slots_v[g // 8, pl.ds((g % 8) * LANES, LANES)] = (
            jnp.where(ok, slot, NOUT + wid))
        carry = carry + tot

    for j in range(nc):
        pltpu.async_copy(anch_hbm.at[vals_v.at[pl.ds(j * 128, 128)]],
                         rows_v.at[pl.ds(j * 128, 128)], sem).wait()

    zeros = jnp.zeros((LANES,), jnp.int32)

    def wcol(g, _):
        rid = lanes + g * LANES
        v = vals_v[pl.ds(g * LANES, LANES)].astype(jnp.float32)
        plsc.store_scatter(rows_v, [rid, zeros], v)
        return 0

    lax.fori_loop(0, ng, wcol, 0)

    for j in range(nc):
        pltpu.async_copy(rows_v.at[pl.ds(j * 128, 128)],
                         out_hbm.at[slots_v.at[j]], sem).wait()


def _finalize(keep, vals, anch16, wprefix, n_pad):
    ch = n_pad // NWORK
    mesh = plsc.VectorSubcoreMesh(core_axis_name="c", subcore_axis_name="s")
    out = pl.kernel(
        functools.partial(_fin_kernel, n_pad=n_pad),
        out_type=jax.ShapeDtypeStruct((NOUT + NWORK, 16), jnp.float32),
        mesh=mesh,
        compiler_params=pltpu.CompilerParams(needs_layout_passes=False,
                                             use_tc_tiling_on_sc=False),
        scratch_types=[
            pltpu.VMEM((ch,), jnp.float32),
            pltpu.VMEM((ch,), jnp.int32),
            pltpu.VMEM((ch // 128, 128), jnp.int32),
            pltpu.VMEM((ch, 16), jnp.float32),
            pltpu.VMEM((NWORK,), jnp.float32),
            pltpu.VMEM((LANES,), jnp.float32),
            pltpu.SemaphoreType.DMA,
        ],
    )(keep, vals, anch16, wprefix)
    return out


def kernel(anchors, scores):
    n = anchors.shape[0]
    grain = BLK * NWORK
    n_pad = ((n + grain - 1) // grain) * grain

    scores_fg = scores.reshape(-1, 2)[:, 1]
    top_scores_idx = jnp.argsort(scores_fg)
    top_anchors = anchors[top_scores_idx]
    top_scores = top_anchors[:, 1]
    order = jnp.argsort(-top_scores)
    bs = top_anchors[order]
    bs_pad = jnp.concatenate(
        [bs, jnp.full((n_pad - n, 4), PAD_COORD, jnp.float32)], axis=0)

    keep, bsum = _run_nms(bs_pad, n_pad)

    nblk = n_pad // BLK
    csum = jnp.cumsum(bsum)
    count = csum[-1].astype(jnp.int32)
    blocks_per_w = nblk // NWORK
    wprefix = jnp.concatenate(
        [jnp.zeros((1,), jnp.float32),
         csum[blocks_per_w - 1:-1:blocks_per_w]])

    vals = jnp.concatenate(
        [order.astype(jnp.int32), jnp.zeros((n_pad - n,), jnp.int32)])
    anch16 = jnp.pad(anchors, ((0, 0), (1, 11)))

    out = _finalize(keep, vals, anch16, wprefix, n_pad)

    valid = jnp.arange(NMS_FILTER) < count
    col0 = out[:NMS_FILTER, 0].astype(jnp.int32)
    nms_idx = jnp.where(valid, col0, col0[0])
    boxes = jnp.where(valid[:, None], out[:NMS_FILTER, 1:5], out[0:1, 1:5])
    return boxes, nms_idx


# speedup vs baseline: 1.6214x; 1.0071x over previous
import functools

import jax
import jax.numpy as jnp
from jax import lax
from jax.experimental import pallas as pl
from jax.experimental.pallas import tpu as pltpu
from jax.experimental.pallas import tpu_sc as plsc

IOU_THRESHOLD = 0.6
NMS_FILTER = 2000
BLK = 128
PAD_COORD = -1.0e6

NWORK = 32
LANES = 16
NOUT = 2048


def _nms_block_kernel(x1_ref, y1_ref, x2_ref, y2_ref,
                      keep_ref, bsum_ref, removed_ref, m_ref, suff_ref,
                      nblk: int):
    b = pl.program_id(0)

    @pl.when(b == 0)
    def _init():
        removed_ref[...] = jnp.zeros_like(removed_ref)

        def sm_body(i, carry):
            c = nblk - 1 - i
            new = jnp.maximum(jnp.max(y2_ref[pl.ds(c, 1), :]), carry)
            suff_ref[c] = new
            return new

        lax.fori_loop(0, nblk, sm_body, jnp.float32(-3e38))

    cx1 = x1_ref[pl.ds(b, 1), :]
    cy1 = y1_ref[pl.ds(b, 1), :]
    cx2 = x2_ref[pl.ds(b, 1), :]
    cy2 = y2_ref[pl.ds(b, 1), :]
    rx1 = jnp.broadcast_to(cx1, (BLK, BLK)).T
    ry1 = jnp.broadcast_to(cy1, (BLK, BLK)).T
    rx2 = jnp.broadcast_to(cx2, (BLK, BLK)).T
    ry2 = jnp.broadcast_to(cy2, (BLK, BLK)).T
    rarea = (rx2 - rx1) * (ry2 - ry1)

    def iou_gt(ccx1, ccy1, ccx2, ccy2):
        ix1 = jnp.maximum(rx1, ccx1)
        iy1 = jnp.maximum(ry1, ccy1)
        ix2 = jnp.minimum(rx2, ccx2)
        iy2 = jnp.minimum(ry2, ccy2)
        inter = jnp.maximum(ix2 - ix1, 0.0) * jnp.maximum(iy2 - iy1, 0.0)
        carea = (ccx2 - ccx1) * (ccy2 - ccy1)
        iou = inter / (rarea + carea - inter + 1e-9)
        return (iou > IOU_THRESHOLD).astype(jnp.float32)

    m = iou_gt(cx1, cy1, cx2, cy2)
    col_ids = lax.broadcasted_iota(jnp.int32, (BLK, BLK), 1)
    row_ids = lax.broadcasted_iota(jnp.int32, (BLK, BLK), 0)
    m_ref[...] = m * (col_ids > row_ids).astype(jnp.float32)

    lane = lax.broadcasted_iota(jnp.int32, (1, BLK), 1)
    keep0 = 1.0 - removed_ref[pl.ds(b, 1), :]

    def body(i, keep):
        mrow = m_ref[pl.ds(i, 1), :]
        keep_i = jnp.sum(jnp.where(lane == i, keep, 0.0))
        return keep * (1.0 - mrow * keep_i)

    CHUNK = 16

    def chunk_body(k, keep):
        base = pl.multiple_of(k * CHUNK, CHUNK)
        flag = jnp.max(m_ref[pl.ds(base, CHUNK), :])
        return lax.cond(
            flag > 0.0,
            lambda kp: lax.fori_loop(base, base + CHUNK, body, kp),
            lambda kp: kp,
            keep)

    keep = lax.fori_loop(0, BLK // CHUNK, chunk_body, keep0)
    keep_real = keep * (cy1 > -1.0e5).astype(jnp.float32)
    keep_ref[pl.ds(b, 1), :] = keep_real
    bsum_ref[pl.ds(b, 1), :] = jnp.broadcast_to(jnp.sum(keep_real), (1, BLK))

    keep_rows = jnp.broadcast_to(keep, (BLK, BLK)).T
    min_y1_b = jnp.min(cy1)

    c_hi = lax.while_loop(
        lambda c: (c < nblk) & (suff_ref[c] > min_y1_b),
        lambda c: c + 1, b + 1)

    def cross(c, _):
        ccx1 = x1_ref[pl.ds(c, 1), :]
        ccy1 = y1_ref[pl.ds(c, 1), :]
        ccx2 = x2_ref[pl.ds(c, 1), :]
        ccy2 = y2_ref[pl.ds(c, 1), :]
        mc = iou_gt(ccx1, ccy1, ccx2, ccy2) * keep_rows
        sup = jnp.max(mc, axis=0, keepdims=True)
        removed_ref[pl.ds(c, 1), :] = jnp.maximum(
            removed_ref[pl.ds(c, 1), :], sup)
        return 0

    lax.fori_loop(b + 1, c_hi, cross, 0)


def _run_nms(bs, n_pad, interpret=False):
    nblk = n_pad // BLK
    planes = [bs[:, i].reshape(nblk, BLK) for i in range(4)]
    keep, bsum = pl.pallas_call(
        functools.partial(_nms_block_kernel, nblk=nblk),
        grid=(nblk,),
        in_specs=[pl.BlockSpec((nblk, BLK), lambda b: (0, 0))] * 4,
        out_specs=[pl.BlockSpec((nblk, BLK), lambda b: (0, 0))] * 2,
        out_shape=[jax.ShapeDtypeStruct((nblk, BLK), jnp.float32),
                   jax.ShapeDtypeStruct((nblk, BLK), jnp.float32)],
        scratch_shapes=[pltpu.VMEM((nblk, BLK), jnp.float32),
                        pltpu.VMEM((BLK, BLK), jnp.float32),
                        pltpu.SMEM((nblk,), jnp.float32)],
        interpret=interpret,
    )(*planes)
    return keep.reshape(-1), bsum[:, 0]


def _fin_kernel(keep_hbm, vals_hbm, anch_hbm, wpre_hbm, out_hbm,
                keep_v, vals_v, slots_v, rows_v, wpre_v, tmp_v, sem,
                n_pad: int):
    ch = n_pad // NWORK
    ng = ch // LANES
    nc = ch // 128
    c = lax.axis_index("c")
    s = lax.axis_index("s")
    wid = s * 2 + c
    base = wid * ch

    pltpu.sync_copy(keep_hbm.at[pl.ds(base, ch)], keep_v)
    pltpu.sync_copy(vals_hbm.at[pl.ds(base, ch)], vals_v)
    pltpu.sync_copy(wpre_hbm, wpre_v)

    lanes = lax.iota(jnp.int32, LANES)
    last = lanes * 0 + (LANES - 1)
    carry = plsc.load_gather(wpre_v, [lanes * 0 + wid])

    for g in range(ng):
        kv = keep_v[pl.ds(g * LANES, LANES)]
        cur = kv
        for sh in (1, 2, 4, 8):
            tmp_v[pl.ds(0, LANES)] = cur
            sft = plsc.load_gather(tmp_v, [jnp.maximum(lanes - sh, 0)])
            cur = cur + jnp.where(lanes >= sh, sft, 0.0)
        tmp_v[pl.ds(0, LANES)] = cur
        tot = plsc.load_gather(tmp_v, [last])
        slot = (carry + cur - 1.0).astype(jnp.int32)
        ok = (kv > 0.0) & (slot < NOUT)
        slots_v[g // 8, pl.ds((g % 8) * LANES, LANES)] = (
            jnp.where(ok, slot, NOUT + wid))
        carry = carry + tot

    for j in range(nc):
        pltpu.async_copy(anch_hbm.at[vals_v.at[pl.ds(j * 128, 128)]],
                         rows_v.at[pl.ds(j * 128, 128)], sem).wait()

    zeros = jnp.zeros((LANES,), jnp.int32)

    def wcol(g, _):
        rid = lanes + g * LANES
        v = vals_v[pl.ds(g * LANES, LANES)].astype(jnp.float32)
        plsc.store_scatter(rows_v, [rid, zeros], v)
        return 0

    lax.fori_loop(0, ng, wcol, 0)

    for j in range(nc):
        pltpu.async_copy(rows_v.at[pl.ds(j * 128, 128)],
                         out_hbm.at[slots_v.at[j]], sem).wait()


def _finalize(keep, vals, anch16, wprefix, n_pad):
    ch = n_pad // NWORK
    mesh = plsc.VectorSubcoreMesh(core_axis_name="c", subcore_axis_name="s")
    out = pl.kernel(
        functools.partial(_fin_kernel, n_pad=n_pad),
        out_type=jax.ShapeDtypeStruct((NOUT + NWORK, 16), jnp.float32),
        mesh=mesh,
        compiler_params=pltpu.CompilerParams(needs_layout_passes=False,
                                             use_tc_tiling_on_sc=False),
        scratch_types=[
            pltpu.VMEM((ch,), jnp.float32),
            pltpu.VMEM((ch,), jnp.int32),
            pltpu.VMEM((ch // 128, 128), jnp.int32),
            pltpu.VMEM((ch, 16), jnp.float32),
            pltpu.VMEM((NWORK,), jnp.float32),
            pltpu.VMEM((LANES,), jnp.float32),
            pltpu.SemaphoreType.DMA,
        ],
    )(keep, vals, anch16, wprefix)
    return out


def kernel(anchors, scores):
    n = anchors.shape[0]
    grain = BLK * NWORK
    n_pad = ((n + grain - 1) // grain) * grain

    scores_fg = scores.reshape(-1, 2)[:, 1]
    top_scores_idx = jnp.argsort(scores_fg)
    top_scores = anchors[top_scores_idx, 1]
    order = jnp.argsort(-top_scores)
    bs = anchors[top_scores_idx[order]]
    bs_pad = jnp.concatenate(
        [bs, jnp.full((n_pad - n, 4), PAD_COORD, jnp.float32)], axis=0)

    keep, bsum = _run_nms(bs_pad, n_pad)

    nblk = n_pad // BLK
    csum = jnp.cumsum(bsum)
    count = csum[-1].astype(jnp.int32)
    blocks_per_w = nblk // NWORK
    wprefix = jnp.concatenate(
        [jnp.zeros((1,), jnp.float32),
         csum[blocks_per_w - 1:-1:blocks_per_w]])

    vals = jnp.concatenate(
        [order.astype(jnp.int32), jnp.zeros((n_pad - n,), jnp.int32)])
    anch16 = jnp.pad(anchors, ((0, 0), (1, 11)))

    out = _finalize(keep, vals, anch16, wprefix, n_pad)

    valid = jnp.arange(NMS_FILTER) < count
    col0 = out[:NMS_FILTER, 0].astype(jnp.int32)
    nms_idx = jnp.where(valid, col0, col0[0])
    boxes = jnp.where(valid[:, None], out[:NMS_FILTER, 1:5], out[0:1, 1:5])
    return boxes, nms_idx


# speedup vs baseline: 1.8787x; 1.1587x over previous
import functools

import jax
import jax.numpy as jnp
from jax import lax
from jax.experimental import pallas as pl
from jax.experimental.pallas import tpu as pltpu
from jax.experimental.pallas import tpu_sc as plsc

IOU_THRESHOLD = 0.6
NMS_FILTER = 2000
BLK = 128
PAD_COORD = -1.0e6

NWORK = 32
LANES = 16
NOUT = 2048


def _nms_block_kernel(x1_ref, y1_ref, x2_ref, y2_ref,
                      keep_ref, bsum_ref, removed_ref, m_ref, suff_ref,
                      nblk: int):
    b = pl.program_id(0)

    @pl.when(b == 0)
    def _init():
        removed_ref[...] = jnp.zeros_like(removed_ref)

        def sm_body(i, carry):
            c = nblk - 1 - i
            new = jnp.maximum(jnp.max(y2_ref[pl.ds(c, 1), :]), carry)
            suff_ref[c] = new
            return new

        lax.fori_loop(0, nblk, sm_body, jnp.float32(-3e38))

    cx1 = x1_ref[pl.ds(b, 1), :]
    cy1 = y1_ref[pl.ds(b, 1), :]
    cx2 = x2_ref[pl.ds(b, 1), :]
    cy2 = y2_ref[pl.ds(b, 1), :]
    rx1 = jnp.broadcast_to(cx1, (BLK, BLK)).T
    ry1 = jnp.broadcast_to(cy1, (BLK, BLK)).T
    rx2 = jnp.broadcast_to(cx2, (BLK, BLK)).T
    ry2 = jnp.broadcast_to(cy2, (BLK, BLK)).T
    rarea = (rx2 - rx1) * (ry2 - ry1)

    def iou_gt(ccx1, ccy1, ccx2, ccy2):
        ix1 = jnp.maximum(rx1, ccx1)
        iy1 = jnp.maximum(ry1, ccy1)
        ix2 = jnp.minimum(rx2, ccx2)
        iy2 = jnp.minimum(ry2, ccy2)
        inter = jnp.maximum(ix2 - ix1, 0.0) * jnp.maximum(iy2 - iy1, 0.0)
        carea = (ccx2 - ccx1) * (ccy2 - ccy1)
        iou = inter / (rarea + carea - inter + 1e-9)
        return (iou > IOU_THRESHOLD).astype(jnp.float32)

    m = iou_gt(cx1, cy1, cx2, cy2)
    col_ids = lax.broadcasted_iota(jnp.int32, (BLK, BLK), 1)
    row_ids = lax.broadcasted_iota(jnp.int32, (BLK, BLK), 0)
    m_ref[...] = m * (col_ids > row_ids).astype(jnp.float32)

    lane = lax.broadcasted_iota(jnp.int32, (1, BLK), 1)
    keep0 = 1.0 - removed_ref[pl.ds(b, 1), :]

    def body(i, keep):
        mrow = m_ref[pl.ds(i, 1), :]
        keep_i = jnp.sum(jnp.where(lane == i, keep, 0.0))
        return keep * (1.0 - mrow * keep_i)

    CHUNK = 8

    def chunk_body(k, keep):
        base = pl.multiple_of(k * CHUNK, CHUNK)
        flag = jnp.max(m_ref[pl.ds(base, CHUNK), :])
        return lax.cond(
            flag > 0.0,
            lambda kp: lax.fori_loop(base, base + CHUNK, body, kp),
            lambda kp: kp,
            keep)

    keep = lax.fori_loop(0, BLK // CHUNK, chunk_body, keep0)
    keep_real = keep * (cy1 > -1.0e5).astype(jnp.float32)
    keep_ref[pl.ds(b, 1), :] = keep_real
    bsum_ref[pl.ds(b, 1), :] = jnp.broadcast_to(jnp.sum(keep_real), (1, BLK))

    keep_rows = jnp.broadcast_to(keep, (BLK, BLK)).T
    min_y1_b = jnp.min(cy1)

    c_hi = lax.while_loop(
        lambda c: (c < nblk) & (suff_ref[c] > min_y1_b),
        lambda c: c + 1, b + 1)

    def cross(c, _):
        ccx1 = x1_ref[pl.ds(c, 1), :]
        ccy1 = y1_ref[pl.ds(c, 1), :]
        ccx2 = x2_ref[pl.ds(c, 1), :]
        ccy2 = y2_ref[pl.ds(c, 1), :]
        mc = iou_gt(ccx1, ccy1, ccx2, ccy2) * keep_rows
        sup = jnp.max(mc, axis=0, keepdims=True)
        removed_ref[pl.ds(c, 1), :] = jnp.maximum(
            removed_ref[pl.ds(c, 1), :], sup)
        return 0

    lax.fori_loop(b + 1, c_hi, cross, 0)


def _run_nms(bs, n_pad, interpret=False):
    nblk = n_pad // BLK
    planes = [bs[:, i].reshape(nblk, BLK) for i in range(4)]
    keep, bsum = pl.pallas_call(
        functools.partial(_nms_block_kernel, nblk=nblk),
        grid=(nblk,),
        in_specs=[pl.BlockSpec((nblk, BLK), lambda b: (0, 0))] * 4,
        out_specs=[pl.BlockSpec((nblk, BLK), lambda b: (0, 0))] * 2,
        out_shape=[jax.ShapeDtypeStruct((nblk, BLK), jnp.float32),
                   jax.ShapeDtypeStruct((nblk, BLK), jnp.float32)],
        scratch_shapes=[pltpu.VMEM((nblk, BLK), jnp.float32),
                        pltpu.VMEM((BLK, BLK), jnp.float32),
                        pltpu.SMEM((nblk,), jnp.float32)],
        interpret=interpret,
    )(*planes)
    return keep.reshape(-1), bsum[:, 0]


def _fin_kernel(keep_hbm, vals_hbm, anch_hbm, wpre_hbm, out_hbm,
                keep_v, vals_v, slots_v, rows_v, wpre_v, tmp_v, sem,
                n_pad: int):
    ch = n_pad // NWORK
    ng = ch // LANES
    nc = ch // 128
    c = lax.axis_index("c")
    s = lax.axis_index("s")
    wid = s * 2 + c
    base = wid * ch

    pltpu.sync_copy(keep_hbm.at[pl.ds(base, ch)], keep_v)
    pltpu.sync_copy(vals_hbm.at[pl.ds(base, ch)], vals_v)
    pltpu.sync_copy(wpre_hbm, wpre_v)

    lanes = lax.iota(jnp.int32, LANES)
    last = lanes * 0 + (LANES - 1)
    carry = plsc.load_gather(wpre_v, [lanes * 0 + wid])

    for g in range(ng):
        kv = keep_v[pl.ds(g * LANES, LANES)]
        cur = kv
        for sh in (1, 2, 4, 8):
            tmp_v[pl.ds(0, LANES)] = cur
            sft = plsc.load_gather(tmp_v, [jnp.maximum(lanes - sh, 0)])
            cur = cur + jnp.where(lanes >= sh, sft, 0.0)
        tmp_v[pl.ds(0, LANES)] = cur
        tot = plsc.load_gather(tmp_v, [last])
        slot = (carry + cur - 1.0).astype(jnp.int32)
        ok = (kv > 0.0) & (slot < NOUT)
        slots_v[g // 8, pl.ds((g % 8) * LANES, LANES)] = (
            jnp.where(ok, slot, NOUT + wid))
        carry = carry + tot

    for j in range(nc):
        pltpu.async_copy(anch_hbm.at[vals_v.at[pl.ds(j * 128, 128)]],
                         rows_v.at[pl.ds(j * 128, 128)], sem).wait()

    zeros = jnp.zeros((LANES,), jnp.int32)

    def wcol(g, _):
        rid = lanes + g * LANES
        v = vals_v[pl.ds(g * LANES, LANES)].astype(jnp.float32)
        plsc.store_scatter(rows_v, [rid, zeros], v)
        return 0

    lax.fori_loop(0, ng, wcol, 0)

    for j in range(nc):
        pltpu.async_copy(rows_v.at[pl.ds(j * 128, 128)],
                         out_hbm.at[slots_v.at[j]], sem).wait()


def _finalize(keep, vals, anch16, wprefix, n_pad):
    ch = n_pad // NWORK
    mesh = plsc.VectorSubcoreMesh(core_axis_name="c", subcore_axis_name="s")
    out = pl.kernel(
        functools.partial(_fin_kernel, n_pad=n_pad),
        out_type=jax.ShapeDtypeStruct((NOUT + NWORK, 16), jnp.float32),
        mesh=mesh,
        compiler_params=pltpu.CompilerParams(needs_layout_passes=False,
                                             use_tc_tiling_on_sc=False),
        scratch_types=[
            pltpu.VMEM((ch,), jnp.float32),
            pltpu.VMEM((ch,), jnp.int32),
            pltpu.VMEM((ch // 128, 128), jnp.int32),
            pltpu.VMEM((ch, 16), jnp.float32),
            pltpu.VMEM((NWORK,), jnp.float32),
            pltpu.VMEM((LANES,), jnp.float32),
            pltpu.SemaphoreType.DMA,
        ],
    )(keep, vals, anch16, wprefix)
    return out


def kernel(anchors, scores):
    n = anchors.shape[0]
    grain = BLK * NWORK
    n_pad = ((n + grain - 1) // grain) * grain

    scores_fg = scores.reshape(-1, 2)[:, 1]
    top_scores_idx = jnp.argsort(scores_fg)
    top_scores = anchors[top_scores_idx, 1]
    order = jnp.argsort(-top_scores)
    bs = anchors[top_scores_idx[order]]
    bs_pad = jnp.concatenate(
        [bs, jnp.full((n_pad - n, 4), PAD_COORD, jnp.float32)], axis=0)

    keep, bsum = _run_nms(bs_pad, n_pad)

    nblk = n_pad // BLK
    csum = jnp.cumsum(bsum)
    count = csum[-1].astype(jnp.int32)
    blocks_per_w = nblk // NWORK
    wprefix = jnp.concatenate(
        [jnp.zeros((1,), jnp.float32),
         csum[blocks_per_w - 1:-1:blocks_per_w]])

    vals = jnp.concatenate(
        [order.astype(jnp.int32), jnp.zeros((n_pad - n,), jnp.int32)])
    anch16 = jnp.pad(anchors, ((0, 0), (1, 11)))

    out = _finalize(keep, vals, anch16, wprefix, n_pad)

    valid = jnp.arange(NMS_FILTER) < count
    col0 = out[:NMS_FILTER, 0].astype(jnp.int32)
    nms_idx = jnp.where(valid, col0, col0[0])
    boxes = jnp.where(valid[:, None], out[:NMS_FILTER, 1:5], out[0:1, 1:5])
    return boxes, nms_idx


# speedup vs baseline: 2.0353x; 1.0833x over previous
import functools

import jax
import jax.numpy as jnp
from jax import lax
from jax.experimental import pallas as pl
from jax.experimental.pallas import tpu as pltpu
from jax.experimental.pallas import tpu_sc as plsc

IOU_THRESHOLD = 0.6
NMS_FILTER = 2000
BLK = 128
PAD_COORD = -1.0e6

NWORK = 32
LANES = 16
NOUT = 2048


def _nms_block_kernel(x1_ref, y1_ref, x2_ref, y2_ref,
                      keep_ref, bsum_ref, removed_ref, m_ref, suff_ref,
                      nblk: int):
    b = pl.program_id(0)

    @pl.when(b == 0)
    def _init():
        removed_ref[...] = jnp.zeros_like(removed_ref)

        def sm_body(i, carry):
            c = nblk - 1 - i
            new = jnp.maximum(jnp.max(y2_ref[pl.ds(c, 1), :]), carry)
            suff_ref[c] = new
            return new

        lax.fori_loop(0, nblk, sm_body, jnp.float32(-3e38))

    cx1 = x1_ref[pl.ds(b, 1), :]
    cy1 = y1_ref[pl.ds(b, 1), :]
    cx2 = x2_ref[pl.ds(b, 1), :]
    cy2 = y2_ref[pl.ds(b, 1), :]
    rx1 = jnp.broadcast_to(cx1, (BLK, BLK)).T
    ry1 = jnp.broadcast_to(cy1, (BLK, BLK)).T
    rx2 = jnp.broadcast_to(cx2, (BLK, BLK)).T
    ry2 = jnp.broadcast_to(cy2, (BLK, BLK)).T
    rarea = (rx2 - rx1) * (ry2 - ry1)

    def iou_gt(ccx1, ccy1, ccx2, ccy2):
        ix1 = jnp.maximum(rx1, ccx1)
        iy1 = jnp.maximum(ry1, ccy1)
        ix2 = jnp.minimum(rx2, ccx2)
        iy2 = jnp.minimum(ry2, ccy2)
        inter = jnp.maximum(ix2 - ix1, 0.0) * jnp.maximum(iy2 - iy1, 0.0)
        carea = (ccx2 - ccx1) * (ccy2 - ccy1)
        iou = inter / (rarea + carea - inter + 1e-9)
        return (iou > IOU_THRESHOLD).astype(jnp.float32)

    m = iou_gt(cx1, cy1, cx2, cy2)
    col_ids = lax.broadcasted_iota(jnp.int32, (BLK, BLK), 1)
    row_ids = lax.broadcasted_iota(jnp.int32, (BLK, BLK), 0)
    m_ref[...] = m * (col_ids > row_ids).astype(jnp.float32)

    lane = lax.broadcasted_iota(jnp.int32, (1, BLK), 1)
    keep0 = 1.0 - removed_ref[pl.ds(b, 1), :]

    def body(i, keep):
        mrow = m_ref[pl.ds(i, 1), :]
        keep_i = jnp.sum(jnp.where(lane == i, keep, 0.0))
        return keep * (1.0 - mrow * keep_i)

    CHUNK = 4

    def chunk_body(k, keep):
        base = pl.multiple_of(k * CHUNK, CHUNK)
        flag = jnp.max(m_ref[pl.ds(base, CHUNK), :])
        return lax.cond(
            flag > 0.0,
            lambda kp: lax.fori_loop(base, base + CHUNK, body, kp),
            lambda kp: kp,
            keep)

    keep = lax.fori_loop(0, BLK // CHUNK, chunk_body, keep0)
    keep_real = keep * (cy1 > -1.0e5).astype(jnp.float32)
    keep_ref[pl.ds(b, 1), :] = keep_real
    bsum_ref[pl.ds(b, 1), :] = jnp.broadcast_to(jnp.sum(keep_real), (1, BLK))

    keep_rows = jnp.broadcast_to(keep, (BLK, BLK)).T
    min_y1_b = jnp.min(cy1)

    c_hi = lax.while_loop(
        lambda c: (c < nblk) & (suff_ref[c] > min_y1_b),
        lambda c: c + 1, b + 1)

    def cross(c, _):
        ccx1 = x1_ref[pl.ds(c, 1), :]
        ccy1 = y1_ref[pl.ds(c, 1), :]
        ccx2 = x2_ref[pl.ds(c, 1), :]
        ccy2 = y2_ref[pl.ds(c, 1), :]
        mc = iou_gt(ccx1, ccy1, ccx2, ccy2) * keep_rows
        sup = jnp.max(mc, axis=0, keepdims=True)
        removed_ref[pl.ds(c, 1), :] = jnp.maximum(
            removed_ref[pl.ds(c, 1), :], sup)
        return 0

    lax.fori_loop(b + 1, c_hi, cross, 0)


def _run_nms(bs, n_pad, interpret=False):
    nblk = n_pad // BLK
    planes = [bs[:, i].reshape(nblk, BLK) for i in range(4)]
    keep, bsum = pl.pallas_call(
        functools.partial(_nms_block_kernel, nblk=nblk),
        grid=(nblk,),
        in_specs=[pl.BlockSpec((nblk, BLK), lambda b: (0, 0))] * 4,
        out_specs=[pl.BlockSpec((nblk, BLK), lambda b: (0, 0))] * 2,
        out_shape=[jax.ShapeDtypeStruct((nblk, BLK), jnp.float32),
                   jax.ShapeDtypeStruct((nblk, BLK), jnp.float32)],
        scratch_shapes=[pltpu.VMEM((nblk, BLK), jnp.float32),
                        pltpu.VMEM((BLK, BLK), jnp.float32),
                        pltpu.SMEM((nblk,), jnp.float32)],
        interpret=interpret,
    )(*planes)
    return keep.reshape(-1), bsum[:, 0]


def _fin_kernel(keep_hbm, vals_hbm, anch_hbm, wpre_hbm, out_hbm,
                keep_v, vals_v, slots_v, rows_v, wpre_v, tmp_v, sem,
                n_pad: int):
    ch = n_pad // NWORK
    ng = ch // LANES
    nc = ch // 128
    c = lax.axis_index("c")
    s = lax.axis_index("s")
    wid = s * 2 + c
    base = wid * ch

    pltpu.sync_copy(keep_hbm.at[pl.ds(base, ch)], keep_v)
    pltpu.sync_copy(vals_hbm.at[pl.ds(base, ch)], vals_v)
    pltpu.sync_copy(wpre_hbm, wpre_v)

    lanes = lax.iota(jnp.int32, LANES)
    last = lanes * 0 + (LANES - 1)
    carry = plsc.load_gather(wpre_v, [lanes * 0 + wid])

    for g in range(ng):
        kv = keep_v[pl.ds(g * LANES, LANES)]
        cur = kv
        for sh in (1, 2, 4, 8):
            tmp_v[pl.ds(0, LANES)] = cur
            sft = plsc.load_gather(tmp_v, [jnp.maximum(lanes - sh, 0)])
            cur = cur + jnp.where(lanes >= sh, sft, 0.0)
        tmp_v[pl.ds(0, LANES)] = cur
        tot = plsc.load_gather(tmp_v, [last])
        slot = (carry + cur - 1.0).astype(jnp.int32)
        ok = (kv > 0.0) & (slot < NOUT)
        slots_v[g // 8, pl.ds((g % 8) * LANES, LANES)] = (
            jnp.where(ok, slot, NOUT + wid))
        carry = carry + tot

    for j in range(nc):
        pltpu.async_copy(anch_hbm.at[vals_v.at[pl.ds(j * 128, 128)]],
                         rows_v.at[pl.ds(j * 128, 128)], sem).wait()

    zeros = jnp.zeros((LANES,), jnp.int32)

    def wcol(g, _):
        rid = lanes + g * LANES
        v = vals_v[pl.ds(g * LANES, LANES)].astype(jnp.float32)
        plsc.store_scatter(rows_v, [rid, zeros], v)
        return 0

    lax.fori_loop(0, ng, wcol, 0)

    for j in range(nc):
        pltpu.async_copy(rows_v.at[pl.ds(j * 128, 128)],
                         out_hbm.at[slots_v.at[j]], sem).wait()


def _finalize(keep, vals, anch16, wprefix, n_pad):
    ch = n_pad // NWORK
    mesh = plsc.VectorSubcoreMesh(core_axis_name="c", subcore_axis_name="s")
    out = pl.kernel(
        functools.partial(_fin_kernel, n_pad=n_pad),
        out_type=jax.ShapeDtypeStruct((NOUT + NWORK, 16), jnp.float32),
        mesh=mesh,
        compiler_params=pltpu.CompilerParams(needs_layout_passes=False,
                                             use_tc_tiling_on_sc=False),
        scratch_types=[
            pltpu.VMEM((ch,), jnp.float32),
            pltpu.VMEM((ch,), jnp.int32),
            pltpu.VMEM((ch // 128, 128), jnp.int32),
            pltpu.VMEM((ch, 16), jnp.float32),
            pltpu.VMEM((NWORK,), jnp.float32),
            pltpu.VMEM((LANES,), jnp.float32),
            pltpu.SemaphoreType.DMA,
        ],
    )(keep, vals, anch16, wprefix)
    return out


def kernel(anchors, scores):
    n = anchors.shape[0]
    grain = BLK * NWORK
    n_pad = ((n + grain - 1) // grain) * grain

    scores_fg = scores.reshape(-1, 2)[:, 1]
    top_scores_idx = jnp.argsort(scores_fg)
    top_scores = anchors[top_scores_idx, 1]
    order = jnp.argsort(-top_scores)
    bs = anchors[top_scores_idx[order]]
    bs_pad = jnp.concatenate(
        [bs, jnp.full((n_pad - n, 4), PAD_COORD, jnp.float32)], axis=0)

    keep, bsum = _run_nms(bs_pad, n_pad)

    nblk = n_pad // BLK
    csum = jnp.cumsum(bsum)
    count = csum[-1].astype(jnp.int32)
    blocks_per_w = nblk // NWORK
    wprefix = jnp.concatenate(
        [jnp.zeros((1,), jnp.float32),
         csum[blocks_per_w - 1:-1:blocks_per_w]])

    vals = jnp.concatenate(
        [order.astype(jnp.int32), jnp.zeros((n_pad - n,), jnp.int32)])
    anch16 = jnp.pad(anchors, ((0, 0), (1, 11)))

    out = _finalize(keep, vals, anch16, wprefix, n_pad)

    valid = jnp.arange(NMS_FILTER) < count
    col0 = out[:NMS_FILTER, 0].astype(jnp.int32)
    nms_idx = jnp.where(valid, col0, col0[0])
    boxes = jnp.where(valid[:, None], out[:NMS_FILTER, 1:5], out[0:1, 1:5])
    return boxes, nms_idx
